# Initial kernel scaffold; baseline (speedup 1.0000x reference)
import jax
import jax.numpy as jnp
from jax.experimental import pallas as pl


def kernel(user, item_i, item_j, embed_user_weight, embed_item_weight, ui_rows, ui_cols, ui_vals, d_i, d_j):
    raise NotImplementedError("write your pallas kernel here")


# speedup vs baseline: 1.2541x; 1.2541x over previous
import functools

import jax
import jax.numpy as jnp
from jax import lax
from jax.experimental import pallas as pl
from jax.experimental.pallas import tpu as pltpu
from jax.experimental.pallas import tpu_sc as plsc

_NC = 2
_NS = 16
_LANES = 16
_N = 10000
_FDIM = 256
_NEDGE = 160000
_RPT = 320
_HALF = _NS * _RPT
_NPAD = _NC * _HALF
_EPT = _NEDGE // _NS
_ECH = 1000
_NCHUNK = _EPT // _ECH
_G = 128
_CBUF = _ECH + _G + 2 * _LANES
_TRASH = _CBUF - _LANES
_ACC_ROWS = _HALF + 8


def _spmm_body(rows_hbm, cols_hbm, vals_hbm, x_hbm, y_hbm, d_hbm, out_hbm,
               acc, rows_v, cols_v, vals_v, dstc, colc, valc,
               col_stage, dst_stage, gbuf, dbuf):
    c = lax.axis_index("c")
    s = lax.axis_index("s")
    lo = c * _HALF
    lb = s * _RPT
    gbase = lo + lb

    pltpu.sync_copy(d_hbm.at[pl.ds(pl.multiple_of(gbase, 8), _RPT)], dbuf)
    for cs in (0, _G, _RPT - _G):
        pltpu.sync_copy(y_hbm.at[pl.ds(gbase + cs, _G)], gbuf)

        def _scale_rows(b, _):
            d16 = dbuf[pl.ds(cs + b * _LANES, _LANES)]
            for j in range(_LANES):
                dsp = jnp.broadcast_to(d16[j], (_LANES,))
                r = b * _LANES + j
                for h in range(2):
                    for k in range(_FDIM // (2 * _LANES)):
                        sl = pl.ds(k * _LANES, _LANES)
                        gbuf[r, h, sl] = gbuf[r, h, sl] * dsp
            return 0

        lax.fori_loop(0, _G // _LANES, _scale_rows, 0)
        pltpu.sync_copy(gbuf, acc.at[pl.ds(lb + cs, _G)])

    plsc.subcore_barrier()

    trash = jnp.int32(_TRASH) + lax.iota(jnp.int32, _LANES)

    def _chunk(ch, _):
        ebase = pl.multiple_of(s * _EPT + ch * _ECH, 8)
        pltpu.sync_copy(rows_hbm.at[pl.ds(ebase, _ECH)], rows_v)
        pltpu.sync_copy(cols_hbm.at[pl.ds(ebase, _ECH)], cols_v)
        pltpu.sync_copy(vals_hbm.at[pl.ds(ebase, _ECH)], vals_v)

        def _compact(i, off):
            sl = pl.ds(i * _LANES, _LANES)
            r16 = rows_v[sl]
            m = (r16 >= lo) & (r16 < lo + _HALF)
            cum = plsc.cumsum(m.astype(jnp.int32))
            pos = jnp.where(m, off + cum - 1, trash)
            plsc.store_scatter(dstc, [pos], r16 - lo)
            plsc.store_scatter(colc, [pos], cols_v[sl])
            plsc.store_scatter(valc, [pos], vals_v[sl])
            return off + cum[_LANES - 1]

        n = lax.fori_loop(0, _ECH // _LANES, _compact, jnp.int32(0))

        for k in range(_G // _LANES):
            sl = pl.ds(n + k * _LANES, _LANES)
            dstc[sl] = jnp.full((_LANES,), _HALF, jnp.int32)
            colc[sl] = jnp.zeros((_LANES,), jnp.int32)
            valc[sl] = jnp.zeros((_LANES,), jnp.float32)

        def _group(g, _):
            off = g * _G
            for k in range(_G // _LANES):
                src = pl.ds(off + k * _LANES, _LANES)
                dst = pl.ds(k * _LANES, _LANES)
                col_stage[dst] = colc[src]
                dst_stage[dst] = dstc[src]
            pltpu.sync_copy(x_hbm.at[col_stage], gbuf)

            def _scale_edges(b, _):
                v16 = valc[pl.ds(off + b * _LANES, _LANES)]
                for j in range(_LANES):
                    vsp = jnp.broadcast_to(v16[j], (_LANES,))
                    e = b * _LANES + j
                    for h in range(2):
                        for k in range(_FDIM // (2 * _LANES)):
                            sl = pl.ds(k * _LANES, _LANES)
                            gbuf[e, h, sl] = gbuf[e, h, sl] * vsp
                return 0

            lax.fori_loop(0, _G // _LANES, _scale_edges, 0)
            pltpu.sync_copy(gbuf, acc.at[dst_stage], add=True)
            return 0

        n_groups = (n + _G - 1) >> 7
        lax.fori_loop(0, n_groups, _group, 0)
        return 0

    lax.fori_loop(0, _NCHUNK, _chunk, 0)

    plsc.subcore_barrier()

    pltpu.sync_copy(acc.at[pl.ds(lb, _RPT)], out_hbm.at[pl.ds(gbase, _RPT)])


_spmm = functools.partial(
    pl.kernel,
    mesh=plsc.VectorSubcoreMesh(core_axis_name="c", subcore_axis_name="s"),
    out_type=jax.ShapeDtypeStruct((_NPAD, 2, _FDIM // 2), jnp.float32),
    scratch_types=[
        pltpu.VMEM_SHARED((_ACC_ROWS, 2, _FDIM // 2), jnp.float32),
        pltpu.VMEM((_ECH,), jnp.int32),
        pltpu.VMEM((_ECH,), jnp.int32),
        pltpu.VMEM((_ECH,), jnp.float32),
        pltpu.VMEM((_CBUF,), jnp.int32),
        pltpu.VMEM((_CBUF,), jnp.int32),
        pltpu.VMEM((_CBUF,), jnp.float32),
        pltpu.VMEM((_G,), jnp.int32),
        pltpu.VMEM((_G,), jnp.int32),
        pltpu.VMEM((_G, 2, _FDIM // 2), jnp.float32),
        pltpu.VMEM((_RPT,), jnp.float32),
    ],
    compiler_params=pltpu.CompilerParams(needs_layout_passes=False),
)(_spmm_body)


def kernel(user, item_i, item_j, embed_user_weight, embed_item_weight,
           ui_rows, ui_cols, ui_vals, d_i, d_j):
    rows = ui_rows.astype(jnp.int32)
    cols = ui_cols.astype(jnp.int32)
    vals = ui_vals.astype(jnp.float32)
    pad_rows = _NPAD - _N
    u0 = jnp.pad(embed_user_weight, ((0, pad_rows), (0, 0))).reshape(
        _NPAD, 2, _FDIM // 2)
    i0 = jnp.pad(embed_item_weight, ((0, pad_rows), (0, 0))).reshape(
        _NPAD, 2, _FDIM // 2)
    di = jnp.pad(d_i[:, 0], (0, pad_rows))
    dj = jnp.pad(d_j[:, 0], (0, pad_rows))

    g1u = _spmm(rows, cols, vals, i0, u0, di)
    g1i = _spmm(cols, rows, vals, u0, i0, dj)
    g2u = _spmm(rows, cols, vals, g1i, g1u, di)
    g2i = _spmm(cols, rows, vals, g1u, g1i, dj)
    g3u = _spmm(rows, cols, vals, g2i, g2u, di)
    g3i = _spmm(cols, rows, vals, g2u, g2i, dj)

    def _flat(a):
        return a.reshape(_NPAD, _FDIM)[:_N]

    users = jnp.concatenate(
        (embed_user_weight, _flat(g1u), _flat(g2u), _flat(g3u)), axis=-1)
    items = jnp.concatenate(
        (embed_item_weight, _flat(g1i), _flat(g2i), _flat(g3i)), axis=-1)
    return (users, items)


# speedup vs baseline: 2.0290x; 1.6179x over previous
import functools

import jax
import jax.numpy as jnp
from jax import lax
from jax.experimental import pallas as pl
from jax.experimental.pallas import tpu as pltpu
from jax.experimental.pallas import tpu_sc as plsc

_NC = 2
_NS = 16
_LANES = 16
_N = 10000
_FDIM = 256
_NEDGE = 160000
_RPT = 320
_HALF = _NS * _RPT
_NPAD = _NC * _HALF
_EPT = _NEDGE // _NS
_ECH = 1000
_NCHUNK = _EPT // _ECH
_G = 16
_CBUF = _ECH + _G + 2 * _LANES
_TRASH = _CBUF - _LANES
_ACC_ROWS = _HALF + 8


def _spmm_body(rows_hbm, cols_hbm, vals_hbm, x_hbm, y_hbm, d_hbm, out_hbm,
               acc, rows_v, cols_v, vals_v, dstc, colc, valc,
               col_stage, dst_stage, gbuf, dbuf):
    c = lax.axis_index("c")
    s = lax.axis_index("s")
    lo = c * _HALF
    lb = s * _RPT
    gbase = lo + lb

    pltpu.sync_copy(d_hbm.at[pl.ds(pl.multiple_of(gbase, 8), _RPT)], dbuf)
    for cs in (0, _G, _RPT - _G):
        pltpu.sync_copy(y_hbm.at[pl.ds(gbase + cs, _G)], gbuf)

        def _scale_rows(b, _):
            d16 = dbuf[pl.ds(cs + b * _LANES, _LANES)]
            for j in range(_LANES):
                dsp = jnp.broadcast_to(d16[j], (_LANES,))
                r = b * _LANES + j
                for h in range(2):
                    for k in range(_FDIM // (2 * _LANES)):
                        sl = pl.ds(k * _LANES, _LANES)
                        gbuf[r, h, sl] = gbuf[r, h, sl] * dsp
            return 0

        lax.fori_loop(0, _G // _LANES, _scale_rows, 0)
        pltpu.sync_copy(gbuf, acc.at[pl.ds(lb + cs, _G)])

    plsc.subcore_barrier()

    trash = jnp.int32(_TRASH) + lax.iota(jnp.int32, _LANES)

    def _chunk(ch, _):
        ebase = pl.multiple_of(s * _EPT + ch * _ECH, 8)
        pltpu.sync_copy(rows_hbm.at[pl.ds(ebase, _ECH)], rows_v)
        pltpu.sync_copy(cols_hbm.at[pl.ds(ebase, _ECH)], cols_v)
        pltpu.sync_copy(vals_hbm.at[pl.ds(ebase, _ECH)], vals_v)

        def _compact(i, off):
            sl = pl.ds(i * _LANES, _LANES)
            r16 = rows_v[sl]
            m = (r16 >= lo) & (r16 < lo + _HALF)
            cum = plsc.cumsum(m.astype(jnp.int32))
            pos = jnp.where(m, off + cum - 1, trash)
            plsc.store_scatter(dstc, [pos], r16 - lo)
            plsc.store_scatter(colc, [pos], cols_v[sl])
            plsc.store_scatter(valc, [pos], vals_v[sl])
            return off + cum[_LANES - 1]

        n = lax.fori_loop(0, _ECH // _LANES, _compact, jnp.int32(0))

        for k in range(_G // _LANES):
            sl = pl.ds(n + k * _LANES, _LANES)
            dstc[sl] = jnp.full((_LANES,), _HALF, jnp.int32)
            colc[sl] = jnp.zeros((_LANES,), jnp.int32)
            valc[sl] = jnp.zeros((_LANES,), jnp.float32)

        def _group(g, _):
            off = g * _G
            for k in range(_G // _LANES):
                src = pl.ds(off + k * _LANES, _LANES)
                dst = pl.ds(k * _LANES, _LANES)
                col_stage[dst] = colc[src]
                dst_stage[dst] = dstc[src]
            pltpu.sync_copy(x_hbm.at[col_stage], gbuf)

            def _scale_edges(b, _):
                v16 = valc[pl.ds(off + b * _LANES, _LANES)]
                for j in range(_LANES):
                    vsp = jnp.broadcast_to(v16[j], (_LANES,))
                    e = b * _LANES + j
                    for h in range(2):
                        for k in range(_FDIM // (2 * _LANES)):
                            sl = pl.ds(k * _LANES, _LANES)
                            gbuf[e, h, sl] = gbuf[e, h, sl] * vsp
                return 0

            lax.fori_loop(0, _G // _LANES, _scale_edges, 0)
            pltpu.sync_copy(gbuf, acc.at[dst_stage], add=True)
            return 0

        n_groups = (n + _G - 1) >> 4
        lax.fori_loop(0, n_groups, _group, 0)
        return 0

    lax.fori_loop(0, _NCHUNK, _chunk, 0)

    plsc.subcore_barrier()

    pltpu.sync_copy(acc.at[pl.ds(lb, _RPT)], out_hbm.at[pl.ds(gbase, _RPT)])


_spmm = functools.partial(
    pl.kernel,
    mesh=plsc.VectorSubcoreMesh(core_axis_name="c", subcore_axis_name="s"),
    out_type=jax.ShapeDtypeStruct((_NPAD, 2, _FDIM // 2), jnp.float32),
    scratch_types=[
        pltpu.VMEM_SHARED((_ACC_ROWS, 2, _FDIM // 2), jnp.float32),
        pltpu.VMEM((_ECH,), jnp.int32),
        pltpu.VMEM((_ECH,), jnp.int32),
        pltpu.VMEM((_ECH,), jnp.float32),
        pltpu.VMEM((_CBUF,), jnp.int32),
        pltpu.VMEM((_CBUF,), jnp.int32),
        pltpu.VMEM((_CBUF,), jnp.float32),
        pltpu.VMEM((_G,), jnp.int32),
        pltpu.VMEM((_G,), jnp.int32),
        pltpu.VMEM((_G, 2, _FDIM // 2), jnp.float32),
        pltpu.VMEM((_RPT,), jnp.float32),
    ],
    compiler_params=pltpu.CompilerParams(needs_layout_passes=False),
)(_spmm_body)


def kernel(user, item_i, item_j, embed_user_weight, embed_item_weight,
           ui_rows, ui_cols, ui_vals, d_i, d_j):
    rows = ui_rows.astype(jnp.int32)
    cols = ui_cols.astype(jnp.int32)
    vals = ui_vals.astype(jnp.float32)
    pad_rows = _NPAD - _N
    u0 = jnp.pad(embed_user_weight, ((0, pad_rows), (0, 0))).reshape(
        _NPAD, 2, _FDIM // 2)
    i0 = jnp.pad(embed_item_weight, ((0, pad_rows), (0, 0))).reshape(
        _NPAD, 2, _FDIM // 2)
    di = jnp.pad(d_i[:, 0], (0, pad_rows))
    dj = jnp.pad(d_j[:, 0], (0, pad_rows))

    g1u = _spmm(rows, cols, vals, i0, u0, di)
    g1i = _spmm(cols, rows, vals, u0, i0, dj)
    g2u = _spmm(rows, cols, vals, g1i, g1u, di)
    g2i = _spmm(cols, rows, vals, g1u, g1i, dj)
    g3u = _spmm(rows, cols, vals, g2i, g2u, di)
    g3i = _spmm(cols, rows, vals, g2u, g2i, dj)

    def _flat(a):
        return a.reshape(_NPAD, _FDIM)[:_N]

    users = jnp.concatenate(
        (embed_user_weight, _flat(g1u), _flat(g2u), _flat(g3u)), axis=-1)
    items = jnp.concatenate(
        (embed_item_weight, _flat(g1i), _flat(g2i), _flat(g3i)), axis=-1)
    return (users, items)
